# baseline (speedup 1.0000x reference)
import functools

import jax
import jax.numpy as jnp
from jax import lax
from jax.experimental import pallas as pl
from jax.experimental.pallas import tpu as pltpu
from jax.experimental.pallas import tpu_sc as plsc

NC = 2
NS = 16
NW = NC * NS
CHUNK = 128
NB = 4


def _pipeline(table, sblk, dblk, rows, acc, gs, ss, nchunks):
    for j in range(NB):
        pltpu.async_copy(table.at[sblk.at[j]], rows.at[j], gs[j])

    @pl.loop(0, nchunks, step=NB)
    def _(i):
        for j in range(NB):
            pltpu.make_async_copy(table.at[sblk.at[0]],
                                  rows.at[j], gs[j]).wait()
            pltpu.async_copy(rows.at[j], acc.at[dblk.at[i + j]], ss[j],
                             add=True)
        for j in range(NB):
            @pl.when(i + NB + j < nchunks)
            def _():
                pltpu.make_async_copy(rows.at[j], acc.at[dblk.at[0]],
                                      ss[j]).wait()
                pltpu.async_copy(table.at[sblk.at[i + NB + j]],
                                 rows.at[j], gs[j])

    for j in range(NB):
        pltpu.make_async_copy(rows.at[j], acc.at[dblk.at[0]], ss[j]).wait()


def _writeout(acc, out_hbm, c, s, sub_rows, last_rows):
    @pl.when(s < NS - 1)
    def _():
        pltpu.sync_copy(acc.at[pl.ds(s * sub_rows, sub_rows)],
                        out_hbm.at[c, pl.ds(s * sub_rows, sub_rows)])

    @pl.when(s == NS - 1)
    def _():
        pltpu.sync_copy(acc.at[pl.ds((NS - 1) * sub_rows, last_rows)],
                        out_hbm.at[c, pl.ds((NS - 1) * sub_rows, last_rows)])


def _segment_sum_sc(p, eip, zblk, n_nodes, acc_rows, cpw):
    H = p.shape[1]
    sub_rows = acc_rows // NS
    last_rows = n_nodes - (NS - 1) * sub_rows
    mesh = plsc.VectorSubcoreMesh(core_axis_name="c", subcore_axis_name="s")

    @functools.partial(
        pl.kernel,
        out_type=jax.ShapeDtypeStruct((NC, n_nodes, H), jnp.float32),
        mesh=mesh,
        compiler_params=pltpu.CompilerParams(use_tc_tiling_on_sc=False),
        scratch_types=[
            pltpu.VMEM((cpw, CHUNK), jnp.int32),
            pltpu.VMEM((cpw, CHUNK), jnp.int32),
            pltpu.VMEM((NB, CHUNK, H), jnp.float32),
            pltpu.VMEM_SHARED((n_nodes, H), jnp.float32),
            pltpu.VMEM_SHARED((acc_rows, H), jnp.float32),
        ] + [pltpu.SemaphoreType.DMA] * (2 * NB),
    )
    def seg_kernel(p_hbm, e_hbm, z_hbm, out_hbm, sblk, dblk, rows, table,
                   acc, *sems):
        c = lax.axis_index("c")
        s = lax.axis_index("s")
        w = c * NS + s

        pltpu.sync_copy(z_hbm, acc.at[pl.ds(s * sub_rows, sub_rows)])

        @pl.when(s < NS - 1)
        def _():
            pltpu.sync_copy(p_hbm.at[pl.ds(s * sub_rows, sub_rows)],
                            table.at[pl.ds(s * sub_rows, sub_rows)])

        @pl.when(s == NS - 1)
        def _():
            pltpu.sync_copy(p_hbm.at[pl.ds((NS - 1) * sub_rows, last_rows)],
                            table.at[pl.ds((NS - 1) * sub_rows, last_rows)])

        pltpu.sync_copy(e_hbm.at[0, w], sblk)
        pltpu.sync_copy(e_hbm.at[1, w], dblk)
        plsc.subcore_barrier()
        _pipeline(table, sblk, dblk, rows, acc, sems[:NB], sems[NB:], cpw)
        plsc.subcore_barrier()
        _writeout(acc, out_hbm, c, s, sub_rows, last_rows)

    return seg_kernel(p, eip, zblk)


def _segment_sum_wide_sc(x2, eip, zblk, n_nodes, acc_rows, cpw):
    HW = x2.shape[2]
    sub_rows = acc_rows // NS
    last_rows = n_nodes - (NS - 1) * sub_rows
    cp2 = 2 * cpw
    mesh = plsc.VectorSubcoreMesh(core_axis_name="c", subcore_axis_name="s")

    @functools.partial(
        pl.kernel,
        out_type=jax.ShapeDtypeStruct((NC, n_nodes, HW), jnp.float32),
        mesh=mesh,
        compiler_params=pltpu.CompilerParams(use_tc_tiling_on_sc=False),
        scratch_types=[
            pltpu.VMEM((cp2, CHUNK), jnp.int32),
            pltpu.VMEM((cp2, CHUNK), jnp.int32),
            pltpu.VMEM((NB, CHUNK, HW), jnp.float32),
            pltpu.VMEM_SHARED((acc_rows, HW), jnp.float32),
        ] + [pltpu.SemaphoreType.DMA] * (2 * NB),
    )
    def seg_kernel(x2_hbm, e_hbm, z_hbm, out_hbm, sblk, dblk, rows, acc,
                   *sems):
        c = lax.axis_index("c")
        s = lax.axis_index("s")

        pltpu.sync_copy(z_hbm, acc.at[pl.ds(s * sub_rows, sub_rows)])
        pltpu.sync_copy(e_hbm.at[0, 2 * s], sblk.at[pl.ds(0, cpw)])
        pltpu.sync_copy(e_hbm.at[0, 2 * s + 1], sblk.at[pl.ds(cpw, cpw)])
        pltpu.sync_copy(e_hbm.at[1, 2 * s], dblk.at[pl.ds(0, cpw)])
        pltpu.sync_copy(e_hbm.at[1, 2 * s + 1], dblk.at[pl.ds(cpw, cpw)])
        plsc.subcore_barrier()
        table = x2_hbm.at[c]
        _pipeline(table, sblk, dblk, rows, acc, sems[:NB], sems[NB:], cp2)
        plsc.subcore_barrier()
        _writeout(acc, out_hbm, c, s, sub_rows, last_rows)

    return seg_kernel(x2, eip, zblk)


def _layer_dense(h, agg, w1, b1, w2, b2, blk, split_cols):
    n, hw = h.shape
    k = w1.shape[1]
    ho = w2.shape[1]
    gw = hw // NC if split_cols else hw

    def body(h_ref, g_ref, w1_ref, b1_ref, w2_ref, b2_ref, o_ref):
        if split_cols:
            z = h_ref[...] + jnp.concatenate([g_ref[0], g_ref[1]], axis=1)
        else:
            z = h_ref[...] + g_ref[0] + g_ref[1]
        t = jnp.maximum(
            jnp.dot(z, w1_ref[...], preferred_element_type=jnp.float32)
            + b1_ref[...], 0.0)
        o_ref[...] = jnp.dot(
            t, w2_ref[...], preferred_element_type=jnp.float32) + b2_ref[...]

    return pl.pallas_call(
        body,
        grid=(n // blk,),
        in_specs=[pl.BlockSpec((blk, hw), lambda i: (i, 0)),
                  pl.BlockSpec((NC, blk, gw), lambda i: (0, i, 0)),
                  pl.BlockSpec((hw, k), lambda i: (0, 0)),
                  pl.BlockSpec((1, k), lambda i: (0, 0)),
                  pl.BlockSpec((k, ho), lambda i: (0, 0)),
                  pl.BlockSpec((1, ho), lambda i: (0, 0))],
        out_specs=pl.BlockSpec((blk, ho), lambda i: (i, 0)),
        out_shape=jax.ShapeDtypeStruct((n, ho), jnp.float32),
    )(h, agg, w1, b1.reshape(1, k), w2, b2.reshape(1, ho))


def kernel(x, edge_index, W11, b11, W12, b12, W21, b21, W22, b22, W31, b31,
           W32, b32):
    n, d = x.shape
    h = W11.shape[1]
    e = edge_index.shape[1]

    cpw = 4 * (-(-e // (NW * CHUNK * 4)))
    e_pad = cpw * NW * CHUNK
    sub_rows = 8 * (-(-(n + 1) // (NS * 8)))
    acc_rows = NS * sub_rows
    blk = 1000 if n % 1000 == 0 else 8 * (n // 8)

    pad = e_pad - e
    eip = jnp.stack([
        jnp.concatenate([edge_index[0], jnp.zeros((pad,), jnp.int32)]),
        jnp.concatenate([edge_index[1], jnp.full((pad,), n, jnp.int32)]),
    ]).reshape(2, NW, cpw, CHUNK)
    dw = d // NC
    x2 = jnp.stack([x[:, :dw], x[:, dw:]])
    zwide = jnp.zeros((acc_rows // NS, dw), jnp.float32)
    zblk = jnp.zeros((acc_rows // NS, h), jnp.float32)

    g1 = _segment_sum_wide_sc(x2, eip, zwide, n, acc_rows, cpw)
    h1 = _layer_dense(x, g1, W11, b11, W12, b12, blk, split_cols=True)
    g2 = _segment_sum_sc(h1, eip, zblk, n, acc_rows, cpw)
    h2 = _layer_dense(h1, g2, W21, b21, W22, b22, blk, split_cols=False)
    g3 = _segment_sum_sc(h2, eip, zblk, n, acc_rows, cpw)
    return _layer_dense(h2, g3, W31, b31, W32, b32, blk, split_cols=False)


# speedup vs baseline: 1.0156x; 1.0156x over previous
import functools

import jax
import jax.numpy as jnp
from jax import lax
from jax.experimental import pallas as pl
from jax.experimental.pallas import tpu as pltpu
from jax.experimental.pallas import tpu_sc as plsc

NC = 2
NS = 16
NW = NC * NS
CHUNK = 128
NB = 8
NBW = 4


def _pipeline(table, sblk, dblk, rows, acc, gs, ss, nchunks, nb):
    for j in range(nb):
        pltpu.async_copy(table.at[sblk.at[j]], rows.at[j], gs[j])

    @pl.loop(0, nchunks, step=nb)
    def _(i):
        for j in range(nb):
            pltpu.make_async_copy(table.at[sblk.at[0]],
                                  rows.at[j], gs[j]).wait()
            pltpu.async_copy(rows.at[j], acc.at[dblk.at[i + j]], ss[j],
                             add=True)
        for j in range(nb):
            @pl.when(i + nb + j < nchunks)
            def _():
                pltpu.make_async_copy(rows.at[j], acc.at[dblk.at[0]],
                                      ss[j]).wait()
                pltpu.async_copy(table.at[sblk.at[i + nb + j]],
                                 rows.at[j], gs[j])

    for j in range(nb):
        pltpu.make_async_copy(rows.at[j], acc.at[dblk.at[0]], ss[j]).wait()


def _writeout(acc, out_hbm, c, s, sub_rows, last_rows):
    @pl.when(s < NS - 1)
    def _():
        pltpu.sync_copy(acc.at[pl.ds(s * sub_rows, sub_rows)],
                        out_hbm.at[c, pl.ds(s * sub_rows, sub_rows)])

    @pl.when(s == NS - 1)
    def _():
        pltpu.sync_copy(acc.at[pl.ds((NS - 1) * sub_rows, last_rows)],
                        out_hbm.at[c, pl.ds((NS - 1) * sub_rows, last_rows)])


def _segment_sum_sc(p, eip, zblk, n_nodes, acc_rows, cpw):
    H = p.shape[1]
    sub_rows = acc_rows // NS
    last_rows = n_nodes - (NS - 1) * sub_rows
    mesh = plsc.VectorSubcoreMesh(core_axis_name="c", subcore_axis_name="s")

    @functools.partial(
        pl.kernel,
        out_type=jax.ShapeDtypeStruct((NC, n_nodes, H), jnp.float32),
        mesh=mesh,
        compiler_params=pltpu.CompilerParams(use_tc_tiling_on_sc=False),
        scratch_types=[
            pltpu.VMEM((cpw, CHUNK), jnp.int32),
            pltpu.VMEM((cpw, CHUNK), jnp.int32),
            pltpu.VMEM((NB, CHUNK, H), jnp.float32),
            pltpu.VMEM_SHARED((n_nodes, H), jnp.float32),
            pltpu.VMEM_SHARED((acc_rows, H), jnp.float32),
        ] + [pltpu.SemaphoreType.DMA] * (2 * NB),
    )
    def seg_kernel(p_hbm, e_hbm, z_hbm, out_hbm, sblk, dblk, rows, table,
                   acc, *sems):
        c = lax.axis_index("c")
        s = lax.axis_index("s")
        w = c * NS + s

        pltpu.sync_copy(z_hbm, acc.at[pl.ds(s * sub_rows, sub_rows)])

        @pl.when(s < NS - 1)
        def _():
            pltpu.sync_copy(p_hbm.at[pl.ds(s * sub_rows, sub_rows)],
                            table.at[pl.ds(s * sub_rows, sub_rows)])

        @pl.when(s == NS - 1)
        def _():
            pltpu.sync_copy(p_hbm.at[pl.ds((NS - 1) * sub_rows, last_rows)],
                            table.at[pl.ds((NS - 1) * sub_rows, last_rows)])

        pltpu.sync_copy(e_hbm.at[0, w], sblk)
        pltpu.sync_copy(e_hbm.at[1, w], dblk)
        plsc.subcore_barrier()
        _pipeline(table, sblk, dblk, rows, acc, sems[:NB], sems[NB:], cpw,
                  NB)
        plsc.subcore_barrier()
        _writeout(acc, out_hbm, c, s, sub_rows, last_rows)

    return seg_kernel(p, eip, zblk)


def _segment_sum_wide_sc(x2, eip, zblk, n_nodes, acc_rows, cpw):
    HW = x2.shape[2]
    sub_rows = acc_rows // NS
    last_rows = n_nodes - (NS - 1) * sub_rows
    cp2 = 2 * cpw
    mesh = plsc.VectorSubcoreMesh(core_axis_name="c", subcore_axis_name="s")

    @functools.partial(
        pl.kernel,
        out_type=jax.ShapeDtypeStruct((NC, n_nodes, HW), jnp.float32),
        mesh=mesh,
        compiler_params=pltpu.CompilerParams(use_tc_tiling_on_sc=False),
        scratch_types=[
            pltpu.VMEM((cp2, CHUNK), jnp.int32),
            pltpu.VMEM((cp2, CHUNK), jnp.int32),
            pltpu.VMEM((NBW, CHUNK, HW), jnp.float32),
            pltpu.VMEM_SHARED((acc_rows, HW), jnp.float32),
        ] + [pltpu.SemaphoreType.DMA] * (2 * NBW),
    )
    def seg_kernel(x2_hbm, e_hbm, z_hbm, out_hbm, sblk, dblk, rows, acc,
                   *sems):
        c = lax.axis_index("c")
        s = lax.axis_index("s")

        pltpu.sync_copy(z_hbm, acc.at[pl.ds(s * sub_rows, sub_rows)])
        pltpu.sync_copy(e_hbm.at[0, 2 * s], sblk.at[pl.ds(0, cpw)])
        pltpu.sync_copy(e_hbm.at[0, 2 * s + 1], sblk.at[pl.ds(cpw, cpw)])
        pltpu.sync_copy(e_hbm.at[1, 2 * s], dblk.at[pl.ds(0, cpw)])
        pltpu.sync_copy(e_hbm.at[1, 2 * s + 1], dblk.at[pl.ds(cpw, cpw)])
        plsc.subcore_barrier()
        table = x2_hbm.at[c]
        _pipeline(table, sblk, dblk, rows, acc, sems[:NBW], sems[NBW:], cp2,
                  NBW)
        plsc.subcore_barrier()
        _writeout(acc, out_hbm, c, s, sub_rows, last_rows)

    return seg_kernel(x2, eip, zblk)


def _layer_dense(h, agg, w1, b1, w2, b2, blk, split_cols):
    n, hw = h.shape
    k = w1.shape[1]
    ho = w2.shape[1]
    gw = hw // NC if split_cols else hw

    def body(h_ref, g_ref, w1_ref, b1_ref, w2_ref, b2_ref, o_ref):
        if split_cols:
            z = h_ref[...] + jnp.concatenate([g_ref[0], g_ref[1]], axis=1)
        else:
            z = h_ref[...] + g_ref[0] + g_ref[1]
        t = jnp.maximum(
            jnp.dot(z, w1_ref[...], preferred_element_type=jnp.float32)
            + b1_ref[...], 0.0)
        o_ref[...] = jnp.dot(
            t, w2_ref[...], preferred_element_type=jnp.float32) + b2_ref[...]

    return pl.pallas_call(
        body,
        grid=(n // blk,),
        in_specs=[pl.BlockSpec((blk, hw), lambda i: (i, 0)),
                  pl.BlockSpec((NC, blk, gw), lambda i: (0, i, 0)),
                  pl.BlockSpec((hw, k), lambda i: (0, 0)),
                  pl.BlockSpec((1, k), lambda i: (0, 0)),
                  pl.BlockSpec((k, ho), lambda i: (0, 0)),
                  pl.BlockSpec((1, ho), lambda i: (0, 0))],
        out_specs=pl.BlockSpec((blk, ho), lambda i: (i, 0)),
        out_shape=jax.ShapeDtypeStruct((n, ho), jnp.float32),
    )(h, agg, w1, b1.reshape(1, k), w2, b2.reshape(1, ho))


def kernel(x, edge_index, W11, b11, W12, b12, W21, b21, W22, b22, W31, b31,
           W32, b32):
    n, d = x.shape
    h = W11.shape[1]
    e = edge_index.shape[1]

    cpw = 4 * (-(-e // (NW * CHUNK * 4)))
    e_pad = cpw * NW * CHUNK
    sub_rows = 8 * (-(-(n + 1) // (NS * 8)))
    acc_rows = NS * sub_rows
    blk = 1000 if n % 1000 == 0 else 8 * (n // 8)

    pad = e_pad - e
    eip = jnp.stack([
        jnp.concatenate([edge_index[0], jnp.zeros((pad,), jnp.int32)]),
        jnp.concatenate([edge_index[1], jnp.full((pad,), n, jnp.int32)]),
    ]).reshape(2, NW, cpw, CHUNK)
    dw = d // NC
    x2 = jnp.stack([x[:, :dw], x[:, dw:]])
    zwide = jnp.zeros((acc_rows // NS, dw), jnp.float32)
    zblk = jnp.zeros((acc_rows // NS, h), jnp.float32)

    g1 = _segment_sum_wide_sc(x2, eip, zwide, n, acc_rows, cpw)
    h1 = _layer_dense(x, g1, W11, b11, W12, b12, blk, split_cols=True)
    g2 = _segment_sum_sc(h1, eip, zblk, n, acc_rows, cpw)
    h2 = _layer_dense(h1, g2, W21, b21, W22, b22, blk, split_cols=False)
    g3 = _segment_sum_sc(h2, eip, zblk, n, acc_rows, cpw)
    return _layer_dense(h2, g3, W31, b31, W32, b32, blk, split_cols=False)
